# baseline (speedup 1.0000x reference)
import functools
import math

import jax
import jax.numpy as jnp
from jax import lax
from jax.experimental import pallas as pl
from jax.experimental.pallas import tpu as pltpu
from jax.experimental.pallas import tpu_sc as plsc

N_NODES = 10000
N_EDGES = 160000
N_GRAPHS = 500
D_IN = 128
D_EDGE = 16
D_OUT = 32
D_EH = 64
D_PRED = 512
N_MP = 3
N_TS = 2

_NW = 32
_CH = 128
_E_PAD = 163840
_NCH = _E_PAD // (_NW * _CH)
_N_PAD = N_NODES + 112
_ET = 512


def _proj_body(nf, w, b, o):
    r = jnp.maximum(
        jnp.dot(nf[...], w[...], preferred_element_type=jnp.float32) + b[...], 0.0)
    o[...] = jnp.concatenate(
        [r, jnp.zeros((N_NODES, 128 - D_OUT), jnp.float32)], axis=1)


def _proj(node_feats, w, b):
    return pl.pallas_call(
        _proj_body,
        out_shape=jax.ShapeDtypeStruct((N_NODES, 128), jnp.float32),
    )(node_feats, w, b)


def _msg_body(ef, hs, we1, be1, we2, be2, o):
    ew = jnp.maximum(
        jnp.dot(ef[...], we1[...], preferred_element_type=jnp.float32) + be1[...], 0.0)
    w = jnp.dot(ew, we2[...], preferred_element_type=jnp.float32) + be2[...]
    hsv = hs[:, 0:D_OUT]
    acc = hsv[:, 0:1] * w[:, 0:D_OUT]
    for d in range(1, D_OUT):
        acc = acc + hsv[:, d:d + 1] * w[:, d * D_OUT:(d + 1) * D_OUT]
    o[...] = jnp.concatenate(
        [acc, jnp.zeros((acc.shape[0], 128 - D_OUT), jnp.float32)], axis=1)


def _messages(ef_pad, hsrc, we1, be1, we2, be2):
    grid = (_E_PAD // _ET,)
    return pl.pallas_call(
        _msg_body,
        grid=grid,
        in_specs=[
            pl.BlockSpec((_ET, D_EDGE), lambda i: (i, 0)),
            pl.BlockSpec((_ET, 128), lambda i: (i, 0)),
            pl.BlockSpec((D_EDGE, D_EH), lambda i: (0, 0)),
            pl.BlockSpec((1, D_EH), lambda i: (0, 0)),
            pl.BlockSpec((D_EH, D_OUT * D_OUT), lambda i: (0, 0)),
            pl.BlockSpec((1, D_OUT * D_OUT), lambda i: (0, 0)),
        ],
        out_specs=pl.BlockSpec((_ET, 128), lambda i: (i, 0)),
        out_shape=jax.ShapeDtypeStruct((_E_PAD, 128), jnp.float32),
    )(ef_pad, hsrc, we1, be1, we2, be2)


def _sc_gather(h, idx3):
    mesh = plsc.VectorSubcoreMesh(core_axis_name="c", subcore_axis_name="s")

    @functools.partial(
        pl.kernel,
        mesh=mesh,
        out_type=jax.ShapeDtypeStruct((_E_PAD, 128), jnp.float32),
        scratch_types=[
            pltpu.VMEM((_NCH, _CH), jnp.int32),
            pltpu.VMEM((_CH, 128), jnp.float32),
            pltpu.SemaphoreType.DMA,
        ],
    )
    def gk(h_hbm, idx_hbm, out_hbm, idx_v, buf, sem):
        c = lax.axis_index("c")
        s = lax.axis_index("s")
        wid = s * 2 + c
        pltpu.sync_copy(idx_hbm.at[wid], idx_v)
        base = wid * (_NCH * _CH)

        def body(j, carry):
            pltpu.async_copy(h_hbm.at[idx_v.at[j]], buf, sem).wait()
            pltpu.sync_copy(buf, out_hbm.at[pl.ds(base + j * _CH, _CH)])
            return carry

        lax.fori_loop(0, _NCH, body, 0)

    return gk(h, idx3)


def _sc_scatter(msg, idx3, zeros):
    mesh = plsc.VectorSubcoreMesh(core_axis_name="c", subcore_axis_name="s")

    @functools.partial(
        pl.kernel,
        mesh=mesh,
        out_type=jax.ShapeDtypeStruct((2, _N_PAD, 128), jnp.float32),
        scratch_types=[
            pltpu.VMEM((_NCH, _CH), jnp.int32),
            pltpu.VMEM((_CH, 128), jnp.float32),
            pltpu.VMEM_SHARED((_N_PAD, 128), jnp.float32),
            pltpu.SemaphoreType.DMA,
        ],
    )
    def sk(msg_hbm, idx_hbm, z_hbm, out_hbm, idx_v, buf, acc_sh, sem):
        c = lax.axis_index("c")
        s = lax.axis_index("s")
        wid = s * 2 + c

        @pl.when(s == 0)
        def _():
            pltpu.sync_copy(z_hbm, acc_sh)

        plsc.subcore_barrier()
        pltpu.sync_copy(idx_hbm.at[wid], idx_v)
        base = wid * (_NCH * _CH)

        def body(j, carry):
            pltpu.sync_copy(msg_hbm.at[pl.ds(base + j * _CH, _CH)], buf)
            pltpu.sync_copy(buf, acc_sh.at[idx_v.at[j]], add=True)
            return carry

        lax.fori_loop(0, _NCH, body, 0)
        plsc.subcore_barrier()
        rows = _N_PAD // 16
        pltpu.sync_copy(acc_sh.at[pl.ds(s * rows, rows)],
                        out_hbm.at[c].at[pl.ds(s * rows, rows)])

    return sk(msg, idx3, zeros)


def _gru_body(a0, a1, hid, bconv, wihT, whhT, bih, bhh, o):
    m = jnp.maximum(a0[...] + a1[...] + bconv[...], 0.0)
    h = hid[:, 0:D_OUT]
    gi = jnp.dot(m, wihT[...], preferred_element_type=jnp.float32) + bih[...]
    gh = jnp.dot(h, whhT[...], preferred_element_type=jnp.float32) + bhh[...]
    d = D_OUT
    r = jax.nn.sigmoid(gi[:, 0:d] + gh[:, 0:d])
    z = jax.nn.sigmoid(gi[:, d:2 * d] + gh[:, d:2 * d])
    n = jnp.tanh(gi[:, 2 * d:] + r * gh[:, 2 * d:])
    res = (1.0 - z) * n + z * h
    o[...] = jnp.concatenate(
        [res, jnp.zeros((N_NODES, 128 - D_OUT), jnp.float32)], axis=1)


def _gru_nodes(a0, a1, hidden, bconv, wihT, whhT, bih, bhh):
    return pl.pallas_call(
        _gru_body,
        out_shape=jax.ShapeDtypeStruct((N_NODES, 128), jnp.float32),
    )(a0, a1, hidden, bconv, wihT, whhT, bih, bhh)


def _readout_body(h_ref, gidc_ref, gidr_ref,
                  wlg0, wlh0, bl0, wp0, bp0, wih0, whh0, bih0, bhh0,
                  wlg1, wlh1, bl1, wp1, bp1, wih1, whh1, bih1, bhh1,
                  w1, b1, gamma, beta, w2, b2, o_ref):
    h = h_ref[:, 0:D_OUT]
    gidc = gidc_ref[...]
    gidr = gidr_ref[...]
    s_mat = (lax.broadcasted_iota(jnp.int32, (N_NODES, N_GRAPHS), 1)
             == gidc).astype(jnp.float32)
    st_mat = (lax.broadcasted_iota(jnp.int32, (N_GRAPHS, N_NODES), 0)
              == gidr).astype(jnp.float32)

    def dot(a, b):
        return jnp.dot(a, b, preferred_element_type=jnp.float32)

    gf = dot(st_mat, h)
    ts = ((wlg0, wlh0, bl0, wp0, bp0, wih0, whh0, bih0, bhh0),
          (wlg1, wlh1, bl1, wp1, bp1, wih1, whh1, bih1, bhh1))
    d = D_OUT
    for (wlg, wlh, bl, wp, bp, wih, whh, bih, bhh) in ts:
        ctxg = dot(s_mat, jnp.maximum(gf, 0.0))
        z = (jnp.sum(ctxg * wlg[...] + h * wlh[...], axis=1, keepdims=True)
             + bl[...])
        z = jnp.where(z >= 0.0, z, 0.01 * z)
        ez = jnp.exp(z - jnp.max(z))
        denom = dot(st_mat, ez)
        a = ez / dot(s_mat, denom)
        hv = dot(h, wp[...]) + bp[...]
        gr = dot(st_mat, a * hv)
        gr = jnp.where(gr > 0.0, gr, jnp.exp(gr) - 1.0)
        gi = dot(gr, wih[...]) + bih[...]
        gh = dot(gf, whh[...]) + bhh[...]
        rr = jax.nn.sigmoid(gi[:, 0:d] + gh[:, 0:d])
        zz = jax.nn.sigmoid(gi[:, d:2 * d] + gh[:, d:2 * d])
        nn = jnp.tanh(gi[:, 2 * d:] + rr * gh[:, 2 * d:])
        gf = jnp.maximum((1.0 - zz) * nn + zz * gf, 0.0)

    x = jnp.maximum(dot(gf, w1[...]) + b1[...], 0.0)
    x = x * gamma[...] + beta[...]
    o_ref[...] = dot(x, w2[...]) + b2[...]


def _readout(h, gidc, gidr, tsp, w1, b1, gamma, beta, w2, b2):
    args = [h, gidc, gidr]
    for t in range(N_TS):
        args.extend(tsp[t])
    args.extend([w1, b1, gamma, beta, w2, b2])
    return pl.pallas_call(
        _readout_body,
        out_shape=jax.ShapeDtypeStruct((N_GRAPHS, 1), jnp.float32),
    )(*args)


def kernel(node_feats, edge_feats, edge_index, node_graph_ids, params):
    p = params
    f32 = jnp.float32
    pad_e = _E_PAD - N_EDGES

    src = jnp.concatenate([edge_index[0], jnp.zeros((pad_e,), jnp.int32)])
    dst = jnp.concatenate(
        [edge_index[1], jnp.full((pad_e,), N_NODES, jnp.int32)])
    src3 = src.reshape(_NW, _NCH, _CH)
    dst3 = dst.reshape(_NW, _NCH, _CH)
    ef_pad = jnp.concatenate(
        [edge_feats, jnp.zeros((pad_e, D_EDGE), f32)], axis=0)
    zeros_acc = jnp.zeros((_N_PAD, 128), f32)

    row = lambda v: v.reshape(1, -1)
    h = _proj(node_feats, p['W_proj'], row(p['b_proj']))
    hidden = h

    wihT, whhT = p['gru_wih'].T, p['gru_whh'].T
    bih, bhh = row(p['gru_bih']), row(p['gru_bhh'])
    bconv = row(p['b_conv'])
    be1, be2 = row(p['be1']), row(p['be2'])

    for _ in range(N_MP):
        hsrc = _sc_gather(h, src3)
        msg = _messages(ef_pad, hsrc, p['We1'], be1, p['We2'], be2)
        aggp = _sc_scatter(msg, dst3, zeros_acc)
        h = _gru_nodes(aggp[0, :N_NODES, :D_OUT], aggp[1, :N_NODES, :D_OUT],
                       hidden, bconv, wihT, whhT, bih, bhh)
        hidden = h

    tsp = []
    for t in range(N_TS):
        wl = p['rd%d_Wl' % t]
        tsp.append((
            row(wl[0:D_OUT, 0]), row(wl[D_OUT:, 0]), p['rd%d_bl' % t].reshape(1, 1),
            p['rd%d_Wp' % t], row(p['rd%d_bp' % t]),
            p['rd%d_wih' % t].T, p['rd%d_whh' % t].T,
            row(p['rd%d_bih' % t]), row(p['rd%d_bhh' % t])))
    gamma_s = p['bn_gamma'] * f32(1.0 / math.sqrt(1.0 + 1e-5))
    out = _readout(h, node_graph_ids.reshape(N_NODES, 1),
                   node_graph_ids.reshape(1, N_NODES),
                   tsp, p['W1'], row(p['b1']), row(gamma_s), row(p['bn_beta']),
                   p['W2'], row(p['b2']))
    return out


# speedup vs baseline: 2.2517x; 2.2517x over previous
import functools
import math

import jax
import jax.numpy as jnp
from jax import lax
from jax.experimental import pallas as pl
from jax.experimental.pallas import tpu as pltpu
from jax.experimental.pallas import tpu_sc as plsc

N_NODES = 10000
N_EDGES = 160000
N_GRAPHS = 500
D_IN = 128
D_EDGE = 16
D_OUT = 32
D_EH = 64
D_PRED = 512
N_MP = 3
N_TS = 2

_NW = 32
_CH = 128
_E_PAD = 163840
_NCH = _E_PAD // (_NW * _CH)
_N_PAD = N_NODES + 240
_N_PACK = _N_PAD // 4
_ET = 512
_NBUF = 4


def _proj_body(nf, w, b, o):
    r = jnp.maximum(
        jnp.dot(nf[...], w[...], preferred_element_type=jnp.float32) + b[...], 0.0)
    o[...] = jnp.concatenate(
        [r, jnp.zeros((N_NODES, 128 - D_OUT), jnp.float32)], axis=1)


def _proj(node_feats, w, b):
    return pl.pallas_call(
        _proj_body,
        out_shape=jax.ShapeDtypeStruct((N_NODES, 128), jnp.float32),
    )(node_feats, w, b)


def _msg_body(ef, hs, dq, we1, be1, we2, be2, p32, o):
    ew = jnp.maximum(
        jnp.dot(ef[...], we1[...], preferred_element_type=jnp.float32) + be1[...], 0.0)
    w = jnp.dot(ew, we2[...], preferred_element_type=jnp.float32) + be2[...]
    hexp = jnp.dot(hs[:, 0:D_OUT], p32[...], preferred_element_type=jnp.float32)
    x = w * hexp
    x = x[:, 0:512] + x[:, 512:1024]
    x = x[:, 0:256] + x[:, 256:512]
    x = x[:, 0:128] + x[:, 128:256]
    x = x[:, 0:64] + x[:, 64:128]
    acc = x[:, 0:32] + x[:, 32:64]
    lanes = lax.broadcasted_iota(jnp.int32, (_ET, 128), 1)
    sel = (lanes // D_OUT) == dq[...]
    vals4 = jnp.concatenate([acc, acc, acc, acc], axis=1)
    o[...] = jnp.where(sel, vals4, 0.0)


def _messages(ef_pad, hsrc, dstq, we1, be1, we2, be2, p32):
    grid = (_E_PAD // _ET,)
    return pl.pallas_call(
        _msg_body,
        grid=grid,
        in_specs=[
            pl.BlockSpec((_ET, D_EDGE), lambda i: (i, 0)),
            pl.BlockSpec((_ET, 128), lambda i: (i, 0)),
            pl.BlockSpec((_ET, 1), lambda i: (i, 0)),
            pl.BlockSpec((D_EDGE, D_EH), lambda i: (0, 0)),
            pl.BlockSpec((1, D_EH), lambda i: (0, 0)),
            pl.BlockSpec((D_EH, D_OUT * D_OUT), lambda i: (0, 0)),
            pl.BlockSpec((1, D_OUT * D_OUT), lambda i: (0, 0)),
            pl.BlockSpec((D_OUT, D_OUT * D_OUT), lambda i: (0, 0)),
        ],
        out_specs=pl.BlockSpec((_ET, 128), lambda i: (i, 0)),
        out_shape=jax.ShapeDtypeStruct((_E_PAD, 128), jnp.float32),
    )(ef_pad, hsrc, dstq, we1, be1, we2, be2, p32)


def _sc_gather(h, idx3):
    mesh = plsc.VectorSubcoreMesh(core_axis_name="c", subcore_axis_name="s")

    @functools.partial(
        pl.kernel,
        mesh=mesh,
        out_type=jax.ShapeDtypeStruct((_E_PAD, 128), jnp.float32),
        scratch_types=[
            pltpu.VMEM((_NCH, _CH), jnp.int32),
            pltpu.VMEM((_NBUF, _CH, 128), jnp.float32),
            pltpu.SemaphoreType.DMA((_NBUF,)),
            pltpu.SemaphoreType.DMA((_NBUF,)),
        ],
    )
    def gk(h_hbm, idx_hbm, out_hbm, idx_v, buf, gsem, wsem):
        c = lax.axis_index("c")
        s = lax.axis_index("s")
        wid = s * 2 + c
        pltpu.sync_copy(idx_hbm.at[wid], idx_v)
        base = wid * (_NCH * _CH)

        def body(j, carry):
            descs = []
            for b in range(_NBUF):
                descs.append(pltpu.async_copy(
                    h_hbm.at[idx_v.at[j + b]], buf.at[b], gsem.at[b]))
            wdescs = []
            for b in range(_NBUF):
                descs[b].wait()
                wdescs.append(pltpu.async_copy(
                    buf.at[b],
                    out_hbm.at[pl.ds(base + (j + b) * _CH, _CH)],
                    wsem.at[b]))
            for b in range(_NBUF):
                wdescs[b].wait()
            return carry

        lax.fori_loop(0, _NCH // _NBUF, lambda i, c_: body(i * _NBUF, c_), 0,
                      unroll=False)

    return gk(h, idx3)


def _sc_scatter(msg, idx3, zeros):
    mesh = plsc.VectorSubcoreMesh(core_axis_name="c", subcore_axis_name="s")

    @functools.partial(
        pl.kernel,
        mesh=mesh,
        out_type=jax.ShapeDtypeStruct((2, _N_PACK, 128), jnp.float32),
        scratch_types=[
            pltpu.VMEM((_NCH, _CH), jnp.int32),
            pltpu.VMEM((_NBUF, _CH, 128), jnp.float32),
            pltpu.VMEM_SHARED((_N_PACK, 128), jnp.float32),
            pltpu.SemaphoreType.DMA((_NBUF,)),
            pltpu.SemaphoreType.DMA((_NBUF,)),
        ],
    )
    def sk(msg_hbm, idx_hbm, z_hbm, out_hbm, idx_v, buf, acc_sh, rsem, ssem):
        c = lax.axis_index("c")
        s = lax.axis_index("s")
        wid = s * 2 + c

        @pl.when(s == 0)
        def _():
            pltpu.sync_copy(z_hbm, acc_sh)

        plsc.subcore_barrier()
        pltpu.sync_copy(idx_hbm.at[wid], idx_v)
        base = wid * (_NCH * _CH)

        def body(j, carry):
            descs = []
            for b in range(_NBUF):
                descs.append(pltpu.async_copy(
                    msg_hbm.at[pl.ds(base + (j + b) * _CH, _CH)],
                    buf.at[b], rsem.at[b]))
            sdescs = []
            for b in range(_NBUF):
                descs[b].wait()
                sdescs.append(pltpu.async_copy(
                    buf.at[b], acc_sh.at[idx_v.at[j + b]], ssem.at[b],
                    add=True))
            for b in range(_NBUF):
                sdescs[b].wait()
            return carry

        lax.fori_loop(0, _NCH // _NBUF, lambda i, c_: body(i * _NBUF, c_), 0,
                      unroll=False)
        plsc.subcore_barrier()
        rows = _N_PACK // 16
        pltpu.sync_copy(acc_sh.at[pl.ds(s * rows, rows)],
                        out_hbm.at[c].at[pl.ds(s * rows, rows)])

    return sk(msg, idx3, zeros)


def _gru_body(a0, a1, hid, bconv, wihT, whhT, bih, bhh, o):
    m = jnp.maximum(a0[...] + a1[...] + bconv[...], 0.0)
    h = hid[:, 0:D_OUT]
    gi = jnp.dot(m, wihT[...], preferred_element_type=jnp.float32) + bih[...]
    gh = jnp.dot(h, whhT[...], preferred_element_type=jnp.float32) + bhh[...]
    d = D_OUT
    r = jax.nn.sigmoid(gi[:, 0:d] + gh[:, 0:d])
    z = jax.nn.sigmoid(gi[:, d:2 * d] + gh[:, d:2 * d])
    n = jnp.tanh(gi[:, 2 * d:] + r * gh[:, 2 * d:])
    res = (1.0 - z) * n + z * h
    o[...] = jnp.concatenate(
        [res, jnp.zeros((N_NODES, 128 - D_OUT), jnp.float32)], axis=1)


def _gru_nodes(a0, a1, hidden, bconv, wihT, whhT, bih, bhh):
    return pl.pallas_call(
        _gru_body,
        out_shape=jax.ShapeDtypeStruct((N_NODES, 128), jnp.float32),
    )(a0, a1, hidden, bconv, wihT, whhT, bih, bhh)


def _readout_body(h_ref, gidc_ref, gidr_ref,
                  wlg0, wlh0, bl0, wp0, bp0, wih0, whh0, bih0, bhh0,
                  wlg1, wlh1, bl1, wp1, bp1, wih1, whh1, bih1, bhh1,
                  w1, b1, gamma, beta, w2, b2, o_ref):
    h = h_ref[:, 0:D_OUT]
    gidc = gidc_ref[...]
    gidr = gidr_ref[...]
    s_mat = (lax.broadcasted_iota(jnp.int32, (N_NODES, N_GRAPHS), 1)
             == gidc).astype(jnp.float32)
    st_mat = (lax.broadcasted_iota(jnp.int32, (N_GRAPHS, N_NODES), 0)
              == gidr).astype(jnp.float32)

    def dot(a, b):
        return jnp.dot(a, b, preferred_element_type=jnp.float32)

    gf = dot(st_mat, h)
    ts = ((wlg0, wlh0, bl0, wp0, bp0, wih0, whh0, bih0, bhh0),
          (wlg1, wlh1, bl1, wp1, bp1, wih1, whh1, bih1, bhh1))
    d = D_OUT
    for (wlg, wlh, bl, wp, bp, wih, whh, bih, bhh) in ts:
        ctxg = dot(s_mat, jnp.maximum(gf, 0.0))
        z = (jnp.sum(ctxg * wlg[...] + h * wlh[...], axis=1, keepdims=True)
             + bl[...])
        z = jnp.where(z >= 0.0, z, 0.01 * z)
        ez = jnp.exp(z - jnp.max(z))
        denom = dot(st_mat, ez)
        a = ez / dot(s_mat, denom)
        hv = dot(h, wp[...]) + bp[...]
        gr = dot(st_mat, a * hv)
        gr = jnp.where(gr > 0.0, gr, jnp.exp(gr) - 1.0)
        gi = dot(gr, wih[...]) + bih[...]
        gh = dot(gf, whh[...]) + bhh[...]
        rr = jax.nn.sigmoid(gi[:, 0:d] + gh[:, 0:d])
        zz = jax.nn.sigmoid(gi[:, d:2 * d] + gh[:, d:2 * d])
        nn = jnp.tanh(gi[:, 2 * d:] + rr * gh[:, 2 * d:])
        gf = jnp.maximum((1.0 - zz) * nn + zz * gf, 0.0)

    x = jnp.maximum(dot(gf, w1[...]) + b1[...], 0.0)
    x = x * gamma[...] + beta[...]
    o_ref[...] = dot(x, w2[...]) + b2[...]


def _readout(h, gidc, gidr, tsp, w1, b1, gamma, beta, w2, b2):
    args = [h, gidc, gidr]
    for t in range(N_TS):
        args.extend(tsp[t])
    args.extend([w1, b1, gamma, beta, w2, b2])
    return pl.pallas_call(
        _readout_body,
        out_shape=jax.ShapeDtypeStruct((N_GRAPHS, 1), jnp.float32),
    )(*args)


def kernel(node_feats, edge_feats, edge_index, node_graph_ids, params):
    p = params
    f32 = jnp.float32
    pad_e = _E_PAD - N_EDGES

    src = jnp.concatenate([edge_index[0], jnp.zeros((pad_e,), jnp.int32)])
    dst = jnp.concatenate(
        [edge_index[1], jnp.full((pad_e,), N_NODES, jnp.int32)])
    src3 = src.reshape(_NW, _NCH, _CH)
    dst3 = (dst // 4).reshape(_NW, _NCH, _CH)
    dstq = (dst % 4).reshape(_E_PAD, 1)
    ef_pad = jnp.concatenate(
        [edge_feats, jnp.zeros((pad_e, D_EDGE), f32)], axis=0)
    zeros_acc = jnp.zeros((_N_PACK, 128), f32)

    row = lambda v: v.reshape(1, -1)
    h = _proj(node_feats, p['W_proj'], row(p['b_proj']))
    hidden = h

    wihT, whhT = p['gru_wih'].T, p['gru_whh'].T
    bih, bhh = row(p['gru_bih']), row(p['gru_bhh'])
    bconv = row(p['b_conv'])
    be1, be2 = row(p['be1']), row(p['be2'])

    p32 = jnp.repeat(jnp.eye(D_OUT, dtype=f32), D_OUT, axis=1)
    for _ in range(N_MP):
        hsrc = _sc_gather(h, src3)
        msg = _messages(ef_pad, hsrc, dstq, p['We1'], be1, p['We2'], be2, p32)
        aggp = _sc_scatter(msg, dst3, zeros_acc)
        a0 = aggp[0].reshape(_N_PAD, D_OUT)[:N_NODES]
        a1 = aggp[1].reshape(_N_PAD, D_OUT)[:N_NODES]
        h = _gru_nodes(a0, a1, hidden, bconv, wihT, whhT, bih, bhh)
        hidden = h

    tsp = []
    for t in range(N_TS):
        wl = p['rd%d_Wl' % t]
        tsp.append((
            row(wl[0:D_OUT, 0]), row(wl[D_OUT:, 0]), p['rd%d_bl' % t].reshape(1, 1),
            p['rd%d_Wp' % t], row(p['rd%d_bp' % t]),
            p['rd%d_wih' % t].T, p['rd%d_whh' % t].T,
            row(p['rd%d_bih' % t]), row(p['rd%d_bhh' % t])))
    gamma_s = p['bn_gamma'] * f32(1.0 / math.sqrt(1.0 + 1e-5))
    out = _readout(h, node_graph_ids.reshape(N_NODES, 1),
                   node_graph_ids.reshape(1, N_NODES),
                   tsp, p['W1'], row(p['b1']), row(gamma_s), row(p['bn_beta']),
                   p['W2'], row(p['b2']))
    return out
